# Initial kernel scaffold; baseline (speedup 1.0000x reference)
import jax
import jax.numpy as jnp
from jax import lax
from jax.experimental import pallas as pl
from jax.experimental.pallas import tpu as pltpu
from jax.experimental.pallas import tpu_sc as plsc

N = 262144
F = 64
B = 256
NC = 2
NS = 16
NW = NC * NS
SAMP = N // NW
CHUNK = 256
NCHUNK = SAMP // CHUNK
HIST = 2 * F * B
ROW_UNROLL = 8


def _sc_body(x_hbm, g_hbm, h_hbm, out_hbm, x_buf, g_v, h_v, hist, sem0, sem1):
    c = lax.axis_index("c")
    s = lax.axis_index("s")
    wid = s * NC + c
    base = wid * SAMP
    sems = (sem0, sem1)

    def start_x(ci, b):
        pltpu.make_async_copy(
            x_hbm.at[pl.ds(base + ci * CHUNK, CHUNK)], x_buf.at[b], sems[b]
        ).start()

    def wait_x(b):
        pltpu.make_async_copy(
            x_hbm.at[pl.ds(base, CHUNK)], x_buf.at[b], sems[b]
        ).wait()

    start_x(0, 0)
    start_x(1, 1)
    pltpu.sync_copy(g_hbm.at[pl.ds(base, SAMP)], g_v)
    pltpu.sync_copy(h_hbm.at[pl.ds(base, SAMP)], h_v)

    zeros = jnp.zeros((16,), jnp.float32)

    def zero_body(i, carry):
        hist[pl.ds(i * 16, 16)] = zeros
        return carry

    lax.fori_loop(0, HIST // 16, zero_body, 0)

    lane_off = lax.iota(jnp.int32, 16) * B

    def do_row(gi, r, b):
        gidx = jnp.full((16,), gi, jnp.int32)
        gs = plsc.load_gather(g_v, [gidx])
        hs = plsc.load_gather(h_v, [gidx])
        for fg in range(F // 16):
            xv = x_buf[b, r, pl.ds(fg * 16, 16)]
            idx = xv + (lane_off + fg * 16 * B)
            plsc.addupdate_scatter(hist, [idx], gs)
            plsc.addupdate_scatter(hist, [idx + (F * B)], hs)

    def compute_chunk(ci, b):
        def rows_body(r8, carry):
            for u in range(ROW_UNROLL):
                r = r8 * ROW_UNROLL + u
                do_row(ci * CHUNK + r, r, b)
            return carry

        lax.fori_loop(0, CHUNK // ROW_UNROLL, rows_body, 0)

    def step_body(si, carry):
        for b in range(2):
            ci = si * 2 + b
            wait_x(b)
            compute_chunk(ci, b)

            @pl.when(ci + 2 < NCHUNK)
            def _():
                start_x(ci + 2, b)

        return carry

    lax.fori_loop(0, NCHUNK // 2, step_body, 0)

    pltpu.sync_copy(hist, out_hbm.at[wid])


_sc_hist = pl.kernel(
    _sc_body,
    out_type=jax.ShapeDtypeStruct((NW, HIST), jnp.float32),
    mesh=plsc.VectorSubcoreMesh(
        core_axis_name="c", subcore_axis_name="s", num_cores=NC, num_subcores=NS
    ),
    scratch_types=[
        pltpu.VMEM((2, CHUNK, F), jnp.int32),
        pltpu.VMEM((SAMP,), jnp.float32),
        pltpu.VMEM((SAMP,), jnp.float32),
        pltpu.VMEM((HIST,), jnp.float32),
        pltpu.SemaphoreType.DMA,
        pltpu.SemaphoreType.DMA,
    ],
)


def _tc_body(p_ref, gl_ref, hl_ref):
    acc = jnp.sum(p_ref[...], axis=0)
    bi = lax.broadcasted_iota(jnp.int32, (B, B), 0)
    ki = lax.broadcasted_iota(jnp.int32, (B, B), 1)
    m = (bi > ki).astype(jnp.float32)
    gl_ref[...] = lax.dot(acc[:F], m, precision=lax.Precision.HIGHEST)
    hl_ref[...] = lax.dot(acc[F:], m, precision=lax.Precision.HIGHEST)


_tc_finish = pl.pallas_call(
    _tc_body,
    out_shape=(
        jax.ShapeDtypeStruct((F, B), jnp.float32),
        jax.ShapeDtypeStruct((F, B), jnp.float32),
    ),
)


@jax.jit
def kernel(X, gradient, hessian):
    partials = _sc_hist(X, gradient, hessian)
    gl, hl = _tc_finish(partials.reshape(NW, 2 * F, B))
    return (gl[None], hl[None])


# speedup vs baseline: 90.9529x; 90.9529x over previous
import jax
import jax.numpy as jnp
from jax import lax
from jax.experimental import pallas as pl
from jax.experimental.pallas import tpu as pltpu
from jax.experimental.pallas import tpu_sc as plsc

N = 262144
F = 64
B = 256
NC = 2
NS = 16
NW = NC * NS
SAMP = N // NW
CHUNK = 256
NCHUNK = SAMP // CHUNK
HIST = 2 * F * B
ROW_UNROLL = 8


def _sc_body(x_hbm, g_hbm, h_hbm, out_hbm, x_buf, g_v, h_v, hist, sem0, sem1):
    c = lax.axis_index("c")
    s = lax.axis_index("s")
    wid = s * NC + c
    base = wid * SAMP
    sems = (sem0, sem1)

    def start_x(ci, b):
        pltpu.make_async_copy(
            x_hbm.at[pl.ds(base + ci * CHUNK, CHUNK)], x_buf.at[b], sems[b]
        ).start()

    def wait_x(b):
        pltpu.make_async_copy(
            x_hbm.at[pl.ds(base, CHUNK)], x_buf.at[b], sems[b]
        ).wait()

    start_x(0, 0)
    start_x(1, 1)
    pltpu.sync_copy(g_hbm.at[pl.ds(base, SAMP)], g_v)
    pltpu.sync_copy(h_hbm.at[pl.ds(base, SAMP)], h_v)

    zeros = jnp.zeros((16,), jnp.float32)

    def zero_body(i, carry):
        hist[pl.ds(i * 16, 16)] = zeros
        return carry

    lax.fori_loop(0, HIST // 16, zero_body, 0)

    lane_off = lax.iota(jnp.int32, 16) * B

    def do_row(gi, r, b):
        gidx = jnp.full((16,), gi, jnp.int32)
        gs = plsc.load_gather(g_v, [gidx])
        hs = plsc.load_gather(h_v, [gidx])
        for fg in range(F // 16):
            xv = x_buf[b, r, pl.ds(fg * 16, 16)]
            idx = xv + (lane_off + fg * 16 * B)
            plsc.addupdate_scatter(hist, [idx], gs)
            plsc.addupdate_scatter(hist, [idx + (F * B)], hs)

    def compute_chunk(ci, b):
        def rows_body(r8, carry):
            for u in range(ROW_UNROLL):
                r = r8 * ROW_UNROLL + u
                do_row(ci * CHUNK + r, r, b)
            return carry

        lax.fori_loop(0, CHUNK // ROW_UNROLL, rows_body, 0)

    def step_body(si, carry):
        for b in range(2):
            ci = si * 2 + b
            wait_x(b)
            compute_chunk(ci, b)

            @pl.when(ci + 2 < NCHUNK)
            def _():
                start_x(ci + 2, b)

        return carry

    lax.fori_loop(0, NCHUNK // 2, step_body, 0)

    pltpu.sync_copy(hist, out_hbm.at[wid])


_sc_hist = pl.kernel(
    _sc_body,
    out_type=jax.ShapeDtypeStruct((NW, HIST), jnp.float32),
    mesh=plsc.VectorSubcoreMesh(
        core_axis_name="c", subcore_axis_name="s", num_cores=NC, num_subcores=NS
    ),
    compiler_params=pltpu.CompilerParams(needs_layout_passes=False),
    scratch_types=[
        pltpu.VMEM((2, CHUNK, F), jnp.int32),
        pltpu.VMEM((SAMP,), jnp.float32),
        pltpu.VMEM((SAMP,), jnp.float32),
        pltpu.VMEM((HIST,), jnp.float32),
        pltpu.SemaphoreType.DMA,
        pltpu.SemaphoreType.DMA,
    ],
)


def _tc_body(p_ref, gl_ref, hl_ref):
    acc = jnp.sum(p_ref[...], axis=0)
    bi = lax.broadcasted_iota(jnp.int32, (B, B), 0)
    ki = lax.broadcasted_iota(jnp.int32, (B, B), 1)
    m = (bi > ki).astype(jnp.float32)
    gl_ref[...] = lax.dot(acc[:F], m, precision=lax.Precision.HIGHEST)
    hl_ref[...] = lax.dot(acc[F:], m, precision=lax.Precision.HIGHEST)


_tc_finish = pl.pallas_call(
    _tc_body,
    out_shape=(
        jax.ShapeDtypeStruct((F, B), jnp.float32),
        jax.ShapeDtypeStruct((F, B), jnp.float32),
    ),
)


@jax.jit
def kernel(X, gradient, hessian):
    partials = _sc_hist(X, gradient, hessian)
    gl, hl = _tc_finish(partials.reshape(NW, 2 * F, B))
    return (gl[None], hl[None])


# speedup vs baseline: 150.0115x; 1.6493x over previous
import jax
import jax.numpy as jnp
from jax import lax
from jax.experimental import pallas as pl
from jax.experimental.pallas import tpu as pltpu
from jax.experimental.pallas import tpu_sc as plsc

N = 262144
F = 64
B = 256
NC = 2
NS = 16
NW = NC * NS
SAMP = N // NW
CHUNK = 256
NCHUNK = SAMP // CHUNK
HIST = 2 * F * B
ROW_UNROLL = 8


def _sc_body(x_hbm, g_hbm, h_hbm, out_hbm, x_buf, g_v, h_v, hist, sem0, sem1):
    c = lax.axis_index("c")
    s = lax.axis_index("s")
    wid = s * NC + c
    base = wid * SAMP
    sems = (sem0, sem1)

    def start_x(ci, b):
        pltpu.make_async_copy(
            x_hbm.at[pl.ds(base + ci * CHUNK, CHUNK)], x_buf.at[b], sems[b]
        ).start()

    def wait_x(b):
        pltpu.make_async_copy(
            x_hbm.at[pl.ds(base, CHUNK)], x_buf.at[b], sems[b]
        ).wait()

    start_x(0, 0)
    start_x(1, 1)
    pltpu.sync_copy(g_hbm.at[pl.ds(base, SAMP)], g_v)
    pltpu.sync_copy(h_hbm.at[pl.ds(base, SAMP)], h_v)

    zeros = jnp.zeros((16,), jnp.float32)

    def zero_body(i, carry):
        hist[pl.ds(i * 16, 16)] = zeros
        return carry

    lax.fori_loop(0, HIST // 16, zero_body, 0)

    lane_off = lax.iota(jnp.int32, 16) * B

    def do_row(gi, r, b):
        gidx = jnp.full((16,), gi, jnp.int32)
        gs = plsc.load_gather(g_v, [gidx])
        hs = plsc.load_gather(h_v, [gidx])
        for fg in range(F // 16):
            xv = x_buf[b, r, pl.ds(fg * 16, 16)]
            idx = xv + (lane_off + fg * 16 * B)
            plsc.addupdate_scatter(hist, [idx], gs)
            plsc.addupdate_scatter(hist, [idx + (F * B)], hs)

    def compute_chunk(ci, b):
        @plsc.parallel_loop(0, CHUNK, unroll=ROW_UNROLL)
        def _rows(r):
            do_row(ci * CHUNK + r, r, b)

    def step_body(si, carry):
        for b in range(2):
            ci = si * 2 + b
            wait_x(b)
            compute_chunk(ci, b)

            @pl.when(ci + 2 < NCHUNK)
            def _():
                start_x(ci + 2, b)

        return carry

    lax.fori_loop(0, NCHUNK // 2, step_body, 0)

    pltpu.sync_copy(hist, out_hbm.at[wid])


_sc_hist = pl.kernel(
    _sc_body,
    out_type=jax.ShapeDtypeStruct((NW, HIST), jnp.float32),
    mesh=plsc.VectorSubcoreMesh(
        core_axis_name="c", subcore_axis_name="s", num_cores=NC, num_subcores=NS
    ),
    compiler_params=pltpu.CompilerParams(needs_layout_passes=False),
    scratch_types=[
        pltpu.VMEM((2, CHUNK, F), jnp.int32),
        pltpu.VMEM((SAMP,), jnp.float32),
        pltpu.VMEM((SAMP,), jnp.float32),
        pltpu.VMEM((HIST,), jnp.float32),
        pltpu.SemaphoreType.DMA,
        pltpu.SemaphoreType.DMA,
    ],
)


def _tc_body(p_ref, gl_ref, hl_ref):
    acc = jnp.sum(p_ref[...], axis=0)
    bi = lax.broadcasted_iota(jnp.int32, (B, B), 0)
    ki = lax.broadcasted_iota(jnp.int32, (B, B), 1)
    m = (bi > ki).astype(jnp.float32)
    gl_ref[...] = lax.dot(acc[:F], m, precision=lax.Precision.HIGHEST)
    hl_ref[...] = lax.dot(acc[F:], m, precision=lax.Precision.HIGHEST)


_tc_finish = pl.pallas_call(
    _tc_body,
    out_shape=(
        jax.ShapeDtypeStruct((F, B), jnp.float32),
        jax.ShapeDtypeStruct((F, B), jnp.float32),
    ),
)


@jax.jit
def kernel(X, gradient, hessian):
    partials = _sc_hist(X, gradient, hessian)
    gl, hl = _tc_finish(partials.reshape(NW, 2 * F, B))
    return (gl[None], hl[None])
